# baseline (device time: 55066 ns/iter reference)
import jax
import jax.numpy as jnp
from jax import lax
from jax.experimental import pallas as pl
from jax.experimental.pallas import tpu as pltpu

N_DEV = 4
SUB = 4


def _gelu(z):
    return 0.5 * z * (1.0 + jnp.tanh(0.7978845608 * (z + 0.044715 * z * z * z)))


def kernel(A, B):
    M, _ = A.shape
    _, N = B.shape
    Q = M // N_DEV
    H = Q // 2
    Hs = H // SUB

    def body(a_ref, b_ref, out_ref, part_ref, b_bf16,
             cw_send, cw_recv, ccw_send, ccw_recv,
             cw_ag_my, cw_ag, ccw_ag_my, ccw_ag,
             cw_rs_ssem, cw_rs_rsem, ccw_rs_ssem, ccw_rs_rsem,
             cw_ag_ssem, cw_ag_rsem, ccw_ag_ssem, ccw_ag_rsem):
        i = lax.axis_index("i")
        right = lax.rem(i + 1, N_DEV)
        left = lax.rem(i + (N_DEV - 1), N_DEV)

        cfg = {
            "cw": dict(send=cw_send, recv=cw_recv,
                       rs_ssem=cw_rs_ssem, rs_rsem=cw_rs_rsem,
                       ag_my=cw_ag_my, ag=cw_ag,
                       ag_ssem=cw_ag_ssem, ag_rsem=cw_ag_rsem,
                       dev=right, base=0),
            "ccw": dict(send=ccw_send, recv=ccw_recv,
                        rs_ssem=ccw_rs_ssem, rs_rsem=ccw_rs_rsem,
                        ag_my=ccw_ag_my, ag=ccw_ag,
                        ag_ssem=ccw_ag_ssem, ag_rsem=ccw_ag_rsem,
                        dev=left, base=H),
        }

        def rs_add_chunk(d, s):
            if d == "cw":
                return lax.rem(i + (2 * N_DEV - s - 1), N_DEV)
            return lax.rem(i + s + 1, N_DEV)

        def own_chunk(d):
            if d == "cw":
                return lax.rem(i + 1, N_DEV)
            return lax.rem(i + (N_DEV - 1), N_DEV)

        def ag_origin(d, h):
            if d == "cw":
                return lax.rem(i + (N_DEV - h), N_DEV)
            return lax.rem(i + h, N_DEV)

        def part_sub(d, c, j):
            return part_ref[pl.ds(c * Q + cfg[d]["base"] + j * Hs, Hs), :]

        def rs_rdma(d, s, j):
            c = cfg[d]
            return pltpu.make_async_remote_copy(
                src_ref=c["send"].at[s, pl.ds(j * Hs, Hs)],
                dst_ref=c["recv"].at[s, pl.ds(j * Hs, Hs)],
                send_sem=c["rs_ssem"].at[s, j],
                recv_sem=c["rs_rsem"].at[s, j],
                device_id=(c["dev"],),
                device_id_type=pl.DeviceIdType.MESH,
            )

        def ag_rdma(d, h, j):
            c = cfg[d]
            src = c["ag_my"] if h == 0 else c["ag"].at[h - 1]
            return pltpu.make_async_remote_copy(
                src_ref=src.at[pl.ds(j * Hs, Hs)],
                dst_ref=c["ag"].at[h, pl.ds(j * Hs, Hs)],
                send_sem=c["ag_ssem"].at[h, j],
                recv_sem=c["ag_rsem"].at[h, j],
                device_id=(c["dev"],),
                device_id_type=pl.DeviceIdType.MESH,
            )

        barrier = pltpu.get_barrier_semaphore()
        for nbr in (left, right):
            pl.semaphore_signal(
                barrier, inc=1,
                device_id=(nbr,), device_id_type=pl.DeviceIdType.MESH,
            )
        pl.semaphore_wait(barrier, 2)

        b_bf16[...] = b_ref[...].astype(jnp.bfloat16)

        def compute_rows(start, nrows):
            part_ref[pl.ds(start, nrows), :] = jnp.dot(
                a_ref[pl.ds(start, nrows), :].astype(jnp.bfloat16),
                b_bf16[...],
                preferred_element_type=jnp.float32,
            )

        def compute_chunk(c):
            compute_rows(c * Q, Q)

        send_waits = []
        rs = {d: [[None] * SUB for _ in range(N_DEV - 1)] for d in cfg}
        for d in cfg:
            compute_rows(i * Q + cfg[d]["base"], H)
            for j in range(SUB):
                cfg[d]["send"][0, pl.ds(j * Hs, Hs)] = (
                    part_sub(d, i, j).astype(jnp.bfloat16))
                r = rs_rdma(d, 0, j)
                r.start()
                rs[d][0][j] = r
                send_waits.append(r)

        compute_chunk(lax.rem(i + (N_DEV - 1), N_DEV))
        compute_chunk(lax.rem(i + 1, N_DEV))
        compute_chunk(lax.rem(i + 2, N_DEV))

        reduced = {d: [None] * SUB for d in cfg}
        for s in range(N_DEV - 1):
            for j in range(SUB):
                for d in cfg:
                    rs[d][s][j].wait_recv()
                    summed = (
                        cfg[d]["recv"][s, pl.ds(j * Hs, Hs)].astype(jnp.float32)
                        + part_sub(d, rs_add_chunk(d, s), j)
                    )
                    if s < N_DEV - 2:
                        cfg[d]["send"][s + 1, pl.ds(j * Hs, Hs)] = (
                            summed.astype(jnp.bfloat16))
                        nxt = rs_rdma(d, s + 1, j)
                        nxt.start()
                        rs[d][s + 1][j] = nxt
                        send_waits.append(nxt)
                    else:
                        reduced[d][j] = summed

        ag = {d: [[None] * SUB for _ in range(N_DEV - 1)] for d in cfg}
        g_out = {d: [None] * SUB for d in cfg}
        for j in range(SUB):
            for d in cfg:
                g = _gelu(reduced[d][j])
                g_out[d][j] = g
                cfg[d]["ag_my"][pl.ds(j * Hs, Hs), :] = g.astype(jnp.bfloat16)
                r = ag_rdma(d, 0, j)
                r.start()
                ag[d][0][j] = r
                send_waits.append(r)
        for d in cfg:
            q = own_chunk(d)
            for j in range(SUB):
                out_ref[pl.ds(q * Q + cfg[d]["base"] + j * Hs, Hs), :] = (
                    g_out[d][j])

        for h in range(N_DEV - 1):
            for j in range(SUB):
                stores = []
                for d in cfg:
                    ag[d][h][j].wait_recv()
                    if h < N_DEV - 2:
                        nxt = ag_rdma(d, h + 1, j)
                        nxt.start()
                        ag[d][h + 1][j] = nxt
                        send_waits.append(nxt)
                    stores.append(d)
                for d in stores:
                    o = ag_origin(d, h)
                    out_ref[pl.ds(o * Q + cfg[d]["base"] + j * Hs, Hs), :] = (
                        cfg[d]["ag"][h, pl.ds(j * Hs, Hs)].astype(jnp.float32))

        for rdma in send_waits:
            rdma.wait_send()

    return pl.pallas_call(
        body,
        out_shape=jax.ShapeDtypeStruct((M, N), jnp.float32),
        in_specs=[
            pl.BlockSpec(memory_space=pltpu.VMEM),
            pl.BlockSpec(memory_space=pltpu.VMEM),
        ],
        out_specs=pl.BlockSpec(memory_space=pltpu.VMEM),
        scratch_shapes=[
            pltpu.VMEM((M, N), jnp.float32),
            pltpu.VMEM((B.shape[0], N), jnp.bfloat16),
            pltpu.VMEM((N_DEV - 1, H, N), jnp.bfloat16),
            pltpu.VMEM((N_DEV - 1, H, N), jnp.bfloat16),
            pltpu.VMEM((N_DEV - 1, H, N), jnp.bfloat16),
            pltpu.VMEM((N_DEV - 1, H, N), jnp.bfloat16),
            pltpu.VMEM((H, N), jnp.bfloat16),
            pltpu.VMEM((N_DEV - 1, H, N), jnp.bfloat16),
            pltpu.VMEM((H, N), jnp.bfloat16),
            pltpu.VMEM((N_DEV - 1, H, N), jnp.bfloat16),
            pltpu.SemaphoreType.DMA((N_DEV - 1, SUB)),
            pltpu.SemaphoreType.DMA((N_DEV - 1, SUB)),
            pltpu.SemaphoreType.DMA((N_DEV - 1, SUB)),
            pltpu.SemaphoreType.DMA((N_DEV - 1, SUB)),
            pltpu.SemaphoreType.DMA((N_DEV - 1, SUB)),
            pltpu.SemaphoreType.DMA((N_DEV - 1, SUB)),
            pltpu.SemaphoreType.DMA((N_DEV - 1, SUB)),
            pltpu.SemaphoreType.DMA((N_DEV - 1, SUB)),
        ],
        compiler_params=pltpu.CompilerParams(collective_id=0),
    )(A, B)


# device time: 54854 ns/iter; 1.0039x vs baseline; 1.0039x over previous
import jax
import jax.numpy as jnp
from jax import lax
from jax.experimental import pallas as pl
from jax.experimental.pallas import tpu as pltpu

N_DEV = 4
SUB = 2


def _gelu(z):
    return 0.5 * z * (1.0 + jnp.tanh(0.7978845608 * (z + 0.044715 * z * z * z)))


def kernel(A, B):
    M, _ = A.shape
    _, N = B.shape
    Q = M // N_DEV
    H = Q // 2
    Hs = H // SUB

    def body(a_ref, b_ref, out_ref, part_ref, b_bf16,
             cw_send, cw_recv, ccw_send, ccw_recv,
             cw_ag_my, cw_ag, ccw_ag_my, ccw_ag,
             cw_rs_ssem, cw_rs_rsem, ccw_rs_ssem, ccw_rs_rsem,
             cw_ag_ssem, cw_ag_rsem, ccw_ag_ssem, ccw_ag_rsem):
        i = lax.axis_index("i")
        right = lax.rem(i + 1, N_DEV)
        left = lax.rem(i + (N_DEV - 1), N_DEV)

        cfg = {
            "cw": dict(send=cw_send, recv=cw_recv,
                       rs_ssem=cw_rs_ssem, rs_rsem=cw_rs_rsem,
                       ag_my=cw_ag_my, ag=cw_ag,
                       ag_ssem=cw_ag_ssem, ag_rsem=cw_ag_rsem,
                       dev=right, base=0),
            "ccw": dict(send=ccw_send, recv=ccw_recv,
                        rs_ssem=ccw_rs_ssem, rs_rsem=ccw_rs_rsem,
                        ag_my=ccw_ag_my, ag=ccw_ag,
                        ag_ssem=ccw_ag_ssem, ag_rsem=ccw_ag_rsem,
                        dev=left, base=H),
        }

        def rs_add_chunk(d, s):
            if d == "cw":
                return lax.rem(i + (2 * N_DEV - s - 1), N_DEV)
            return lax.rem(i + s + 1, N_DEV)

        def own_chunk(d):
            if d == "cw":
                return lax.rem(i + 1, N_DEV)
            return lax.rem(i + (N_DEV - 1), N_DEV)

        def ag_origin(d, h):
            if d == "cw":
                return lax.rem(i + (N_DEV - h), N_DEV)
            return lax.rem(i + h, N_DEV)

        def part_sub(d, c, j):
            return part_ref[pl.ds(c * Q + cfg[d]["base"] + j * Hs, Hs), :]

        def rs_rdma(d, s, j):
            c = cfg[d]
            return pltpu.make_async_remote_copy(
                src_ref=c["send"].at[s, pl.ds(j * Hs, Hs)],
                dst_ref=c["recv"].at[s, pl.ds(j * Hs, Hs)],
                send_sem=c["rs_ssem"].at[s, j],
                recv_sem=c["rs_rsem"].at[s, j],
                device_id=(c["dev"],),
                device_id_type=pl.DeviceIdType.MESH,
            )

        def ag_rdma(d, h, j):
            c = cfg[d]
            src = c["ag_my"] if h == 0 else c["ag"].at[h - 1]
            return pltpu.make_async_remote_copy(
                src_ref=src.at[pl.ds(j * Hs, Hs)],
                dst_ref=c["ag"].at[h, pl.ds(j * Hs, Hs)],
                send_sem=c["ag_ssem"].at[h, j],
                recv_sem=c["ag_rsem"].at[h, j],
                device_id=(c["dev"],),
                device_id_type=pl.DeviceIdType.MESH,
            )

        barrier = pltpu.get_barrier_semaphore()
        for nbr in (left, right):
            pl.semaphore_signal(
                barrier, inc=1,
                device_id=(nbr,), device_id_type=pl.DeviceIdType.MESH,
            )
        pl.semaphore_wait(barrier, 2)

        b_bf16[...] = b_ref[...].astype(jnp.bfloat16)

        def compute_rows(start, nrows):
            part_ref[pl.ds(start, nrows), :] = jnp.dot(
                a_ref[pl.ds(start, nrows), :].astype(jnp.bfloat16),
                b_bf16[...],
                preferred_element_type=jnp.float32,
            )

        def compute_chunk(c):
            compute_rows(c * Q, Q)

        send_waits = []
        rs = {d: [[None] * SUB for _ in range(N_DEV - 1)] for d in cfg}
        for d in cfg:
            compute_rows(i * Q + cfg[d]["base"], H)
            for j in range(SUB):
                cfg[d]["send"][0, pl.ds(j * Hs, Hs)] = (
                    part_sub(d, i, j).astype(jnp.bfloat16))
                r = rs_rdma(d, 0, j)
                r.start()
                rs[d][0][j] = r
                send_waits.append(r)

        compute_chunk(lax.rem(i + (N_DEV - 1), N_DEV))
        compute_chunk(lax.rem(i + 1, N_DEV))
        compute_chunk(lax.rem(i + 2, N_DEV))

        reduced = {d: [None] * SUB for d in cfg}
        for s in range(N_DEV - 1):
            for j in range(SUB):
                for d in cfg:
                    rs[d][s][j].wait_recv()
                    summed = (
                        cfg[d]["recv"][s, pl.ds(j * Hs, Hs)].astype(jnp.float32)
                        + part_sub(d, rs_add_chunk(d, s), j)
                    )
                    if s < N_DEV - 2:
                        cfg[d]["send"][s + 1, pl.ds(j * Hs, Hs)] = (
                            summed.astype(jnp.bfloat16))
                        nxt = rs_rdma(d, s + 1, j)
                        nxt.start()
                        rs[d][s + 1][j] = nxt
                        send_waits.append(nxt)
                    else:
                        reduced[d][j] = summed

        ag = {d: [[None] * SUB for _ in range(N_DEV - 1)] for d in cfg}
        g_out = {d: [None] * SUB for d in cfg}
        for j in range(SUB):
            for d in cfg:
                g = _gelu(reduced[d][j])
                g_out[d][j] = g
                cfg[d]["ag_my"][pl.ds(j * Hs, Hs), :] = g.astype(jnp.bfloat16)
                r = ag_rdma(d, 0, j)
                r.start()
                ag[d][0][j] = r
                send_waits.append(r)
        for d in cfg:
            q = own_chunk(d)
            for j in range(SUB):
                out_ref[pl.ds(q * Q + cfg[d]["base"] + j * Hs, Hs), :] = (
                    g_out[d][j])

        for h in range(N_DEV - 1):
            for j in range(SUB):
                stores = []
                for d in cfg:
                    ag[d][h][j].wait_recv()
                    if h < N_DEV - 2:
                        nxt = ag_rdma(d, h + 1, j)
                        nxt.start()
                        ag[d][h + 1][j] = nxt
                        send_waits.append(nxt)
                    stores.append(d)
                for d in stores:
                    o = ag_origin(d, h)
                    out_ref[pl.ds(o * Q + cfg[d]["base"] + j * Hs, Hs), :] = (
                        cfg[d]["ag"][h, pl.ds(j * Hs, Hs)].astype(jnp.float32))

        for rdma in send_waits:
            rdma.wait_send()

    return pl.pallas_call(
        body,
        out_shape=jax.ShapeDtypeStruct((M, N), jnp.float32),
        in_specs=[
            pl.BlockSpec(memory_space=pltpu.VMEM),
            pl.BlockSpec(memory_space=pltpu.VMEM),
        ],
        out_specs=pl.BlockSpec(memory_space=pltpu.VMEM),
        scratch_shapes=[
            pltpu.VMEM((M, N), jnp.float32),
            pltpu.VMEM((B.shape[0], N), jnp.bfloat16),
            pltpu.VMEM((N_DEV - 1, H, N), jnp.bfloat16),
            pltpu.VMEM((N_DEV - 1, H, N), jnp.bfloat16),
            pltpu.VMEM((N_DEV - 1, H, N), jnp.bfloat16),
            pltpu.VMEM((N_DEV - 1, H, N), jnp.bfloat16),
            pltpu.VMEM((H, N), jnp.bfloat16),
            pltpu.VMEM((N_DEV - 1, H, N), jnp.bfloat16),
            pltpu.VMEM((H, N), jnp.bfloat16),
            pltpu.VMEM((N_DEV - 1, H, N), jnp.bfloat16),
            pltpu.SemaphoreType.DMA((N_DEV - 1, SUB)),
            pltpu.SemaphoreType.DMA((N_DEV - 1, SUB)),
            pltpu.SemaphoreType.DMA((N_DEV - 1, SUB)),
            pltpu.SemaphoreType.DMA((N_DEV - 1, SUB)),
            pltpu.SemaphoreType.DMA((N_DEV - 1, SUB)),
            pltpu.SemaphoreType.DMA((N_DEV - 1, SUB)),
            pltpu.SemaphoreType.DMA((N_DEV - 1, SUB)),
            pltpu.SemaphoreType.DMA((N_DEV - 1, SUB)),
        ],
        compiler_params=pltpu.CompilerParams(collective_id=0),
    )(A, B)


# device time: 51545 ns/iter; 1.0683x vs baseline; 1.0642x over previous
import jax
import jax.numpy as jnp
from jax import lax
from jax.experimental import pallas as pl
from jax.experimental.pallas import tpu as pltpu

N_DEV = 4
SUB = 2


def _gelu(z):
    return 0.5 * z * (1.0 + jnp.tanh(0.7978845608 * (z + 0.044715 * z * z * z)))


def kernel(A, B):
    M, _ = A.shape
    _, N = B.shape
    Q = M // N_DEV
    H = Q // 2
    Hs = H // SUB

    def body(a_ref, b_ref, out_ref, part_ref, b_bf16,
             cw_send, cw_recv, ccw_send, ccw_recv,
             cw_rs_ssem, cw_rs_rsem, ccw_rs_ssem, ccw_rs_rsem,
             cw_ag_ssem, cw_ag_rsem, ccw_ag_ssem, ccw_ag_rsem):
        i = lax.axis_index("i")
        right = lax.rem(i + 1, N_DEV)
        left = lax.rem(i + (N_DEV - 1), N_DEV)

        cfg = {
            "cw": dict(send=cw_send, recv=cw_recv,
                       rs_ssem=cw_rs_ssem, rs_rsem=cw_rs_rsem,
                       ag_ssem=cw_ag_ssem, ag_rsem=cw_ag_rsem,
                       dev=right, base=0),
            "ccw": dict(send=ccw_send, recv=ccw_recv,
                        rs_ssem=ccw_rs_ssem, rs_rsem=ccw_rs_rsem,
                        ag_ssem=ccw_ag_ssem, ag_rsem=ccw_ag_rsem,
                        dev=left, base=H),
        }

        def rs_add_chunk(d, s):
            if d == "cw":
                return lax.rem(i + (2 * N_DEV - s - 1), N_DEV)
            return lax.rem(i + s + 1, N_DEV)

        def own_chunk(d):
            if d == "cw":
                return lax.rem(i + 1, N_DEV)
            return lax.rem(i + (N_DEV - 1), N_DEV)

        def ag_send_chunk(d, h):
            if d == "cw":
                return lax.rem(i + (N_DEV + 1 - h), N_DEV)
            return lax.rem(i + (N_DEV - 1 + h), N_DEV)

        def part_sub(d, c, j):
            return part_ref[pl.ds(c * Q + cfg[d]["base"] + j * Hs, Hs), :]

        def rs_rdma(d, s, j):
            c = cfg[d]
            if s == 0:
                src = part_ref.at[
                    pl.ds(i * Q + c["base"] + j * Hs, Hs)]
            else:
                src = c["send"].at[s, pl.ds(j * Hs, Hs)]
            return pltpu.make_async_remote_copy(
                src_ref=src,
                dst_ref=c["recv"].at[s, pl.ds(j * Hs, Hs)],
                send_sem=c["rs_ssem"].at[s, j],
                recv_sem=c["rs_rsem"].at[s, j],
                device_id=(c["dev"],),
                device_id_type=pl.DeviceIdType.MESH,
            )

        def ag_rdma(d, h, j):
            c = cfg[d]
            rows = pl.ds(ag_send_chunk(d, h) * Q + c["base"] + j * Hs, Hs)
            return pltpu.make_async_remote_copy(
                src_ref=out_ref.at[rows],
                dst_ref=out_ref.at[rows],
                send_sem=c["ag_ssem"].at[h, j],
                recv_sem=c["ag_rsem"].at[h, j],
                device_id=(c["dev"],),
                device_id_type=pl.DeviceIdType.MESH,
            )

        barrier = pltpu.get_barrier_semaphore()
        for nbr in (left, right):
            pl.semaphore_signal(
                barrier, inc=1,
                device_id=(nbr,), device_id_type=pl.DeviceIdType.MESH,
            )
        pl.semaphore_wait(barrier, 2)

        b_bf16[...] = b_ref[...].astype(jnp.bfloat16)

        def mm_half(d, c):
            start = c * Q + cfg[d]["base"]
            part_ref[pl.ds(start, H), :] = jnp.dot(
                a_ref[pl.ds(start, H), :].astype(jnp.bfloat16),
                b_bf16[...],
                preferred_element_type=jnp.float32,
            ).astype(jnp.bfloat16)

        send_waits = []
        rs = {d: [[None] * SUB for _ in range(N_DEV - 1)] for d in cfg}
        ag = {d: [[None] * SUB for _ in range(N_DEV - 1)] for d in cfg}

        for d in cfg:
            mm_half(d, i)
            for j in range(SUB):
                r = rs_rdma(d, 0, j)
                r.start()
                rs[d][0][j] = r
                send_waits.append(r)
        mm_half("cw", rs_add_chunk("cw", 0))
        mm_half("ccw", rs_add_chunk("ccw", 0))

        mm_sched = {
            0: [("cw", rs_add_chunk("cw", 1)), ("ccw", rs_add_chunk("ccw", 1))],
            1: [("cw", rs_add_chunk("cw", 2)), ("ccw", rs_add_chunk("ccw", 2))],
        }

        for s in range(N_DEV - 1):
            last = s == N_DEV - 2
            for j in range(SUB):
                for d in cfg:
                    rs[d][s][j].wait_recv()
                    summed = (
                        cfg[d]["recv"][s, pl.ds(j * Hs, Hs)].astype(jnp.float32)
                        + part_sub(d, rs_add_chunk(d, s), j).astype(jnp.float32)
                    )
                    if not last:
                        cfg[d]["send"][s + 1, pl.ds(j * Hs, Hs)] = (
                            summed.astype(jnp.bfloat16))
                        nxt = rs_rdma(d, s + 1, j)
                        nxt.start()
                        rs[d][s + 1][j] = nxt
                        send_waits.append(nxt)
                    else:
                        q = own_chunk(d)
                        out_ref[pl.ds(q * Q + cfg[d]["base"] + j * Hs, Hs),
                                :] = _gelu(summed).astype(jnp.bfloat16)
                        r = ag_rdma(d, 0, j)
                        r.start()
                        ag[d][0][j] = r
                        send_waits.append(r)
                if not last and j < len(mm_sched.get(s, [])):
                    dd, cc = mm_sched[s][j]
                    mm_half(dd, cc)

        for h in range(N_DEV - 1):
            for j in range(SUB):
                for d in cfg:
                    ag[d][h][j].wait_recv()
                    if h < N_DEV - 2:
                        nxt = ag_rdma(d, h + 1, j)
                        nxt.start()
                        ag[d][h + 1][j] = nxt
                        send_waits.append(nxt)

        for rdma in send_waits:
            rdma.wait_send()

    return pl.pallas_call(
        body,
        out_shape=jax.ShapeDtypeStruct((M, N), jnp.bfloat16),
        in_specs=[
            pl.BlockSpec(memory_space=pltpu.VMEM),
            pl.BlockSpec(memory_space=pltpu.VMEM),
        ],
        out_specs=pl.BlockSpec(memory_space=pltpu.VMEM),
        scratch_shapes=[
            pltpu.VMEM((M, N), jnp.bfloat16),
            pltpu.VMEM((B.shape[0], N), jnp.bfloat16),
            pltpu.VMEM((N_DEV - 1, H, N), jnp.bfloat16),
            pltpu.VMEM((N_DEV - 1, H, N), jnp.bfloat16),
            pltpu.VMEM((N_DEV - 1, H, N), jnp.bfloat16),
            pltpu.VMEM((N_DEV - 1, H, N), jnp.bfloat16),
            pltpu.SemaphoreType.DMA((N_DEV - 1, SUB)),
            pltpu.SemaphoreType.DMA((N_DEV - 1, SUB)),
            pltpu.SemaphoreType.DMA((N_DEV - 1, SUB)),
            pltpu.SemaphoreType.DMA((N_DEV - 1, SUB)),
            pltpu.SemaphoreType.DMA((N_DEV - 1, SUB)),
            pltpu.SemaphoreType.DMA((N_DEV - 1, SUB)),
            pltpu.SemaphoreType.DMA((N_DEV - 1, SUB)),
            pltpu.SemaphoreType.DMA((N_DEV - 1, SUB)),
        ],
        compiler_params=pltpu.CompilerParams(collective_id=0),
    )(A, B)
